# Initial kernel scaffold; baseline (speedup 1.0000x reference)
import functools

import jax
import jax.numpy as jnp
from jax import lax
from jax.experimental import pallas as pl
from jax.experimental.pallas import tpu as pltpu
from jax.experimental.pallas import tpu_sc as plsc

N = 100000
E = 1600000
FC = 200000

NC = 2
NS = 16
NW = NC * NS

NPAD = 100096
SL = NPAD // NS
RB = NPAD // 128

ECH = 16
EPAD = 1638400
ER = EPAD // 128
ERT = ER // NW
ENC = ERT // ECH

FCH = 2
FPAD = 204800
FR = FPAD // 128
FRT = FR // NW
FNC = FRT // FCH

_MESH = plsc.VectorSubcoreMesh(core_axis_name="c", subcore_axis_name="s")
_F32 = jnp.float32
_I32 = jnp.int32


def _fill(ref, n, value):
    def body(i, carry):
        ref[pl.ds(i * 16, 16)] = jnp.full((16,), value, _F32)
        return carry
    lax.fori_loop(0, n // 16, body, 0)


def _rsqrt_sc(nn):
    i = plsc.bitcast(nn, _I32)
    i = jnp.int32(0x5F3759DF) - lax.shift_right_logical(i, 1)
    y = plsc.bitcast(i, _F32)
    for _ in range(3):
        y = y * (1.5 - 0.5 * nn * y * y)
    return y


def _p_body(row_h, deg_o, rowv, onesv, zerov, sp_deg, sem_s):
    c = lax.axis_index("c")
    s = lax.axis_index("s")
    w = c * NS + s
    off = s * SL
    _fill(zerov, SL, 0.0)
    _fill(onesv, 128, 1.0)
    pltpu.sync_copy(zerov, sp_deg.at[pl.ds(off, SL)])
    plsc.subcore_barrier()

    ebase = w * ERT

    def chunk(cc, carry):
        r0 = ebase + cc * ECH
        pltpu.sync_copy(row_h.at[pl.ds(r0, ECH)], rowv)
        descs = [
            pltpu.async_copy(onesv, sp_deg.at[rowv.at[j]], sem_s, add=True)
            for j in range(ECH)
        ]
        for d in descs:
            d.wait()
        return carry

    lax.fori_loop(0, ENC, chunk, 0)
    plsc.subcore_barrier()
    pltpu.sync_copy(sp_deg.at[pl.ds(off, SL)], deg_o.at[c, pl.ds(off, SL)])


_p_call = functools.partial(
    pl.kernel,
    out_type=jax.ShapeDtypeStruct((NC, NPAD), _F32),
    mesh=_MESH,
    scratch_types=[
        pltpu.VMEM((ECH, 128), _I32),
        pltpu.VMEM((128,), _F32),
        pltpu.VMEM((SL,), _F32),
        pltpu.VMEM_SHARED((NPAD,), _F32),
        pltpu.SemaphoreType.DMA,
    ],
)(_p_body)


def _a_body(col_h, row_h, fa_h, fb_h, fc_h, px_h, py_h, pz_h, z_h,
            t_o, nxo, nyo, nzo,
            colv, rowv, zg, fav, fbv, fcv,
            gax, gay, gaz, gbx, gby, gbz, gcx, gcy, gcz,
            vnx, vny, vnz, zerov,
            spx, spy, spz, spzz, sp_t, sp_nx, sp_ny, sp_nz,
            sem_g, sem_s):
    c = lax.axis_index("c")
    s = lax.axis_index("s")
    w = c * NS + s
    off = s * SL

    _fill(zerov, SL, 0.0)
    pltpu.sync_copy(zerov, sp_t.at[pl.ds(off, SL)])
    pltpu.sync_copy(zerov, sp_nx.at[pl.ds(off, SL)])
    pltpu.sync_copy(zerov, sp_ny.at[pl.ds(off, SL)])
    pltpu.sync_copy(zerov, sp_nz.at[pl.ds(off, SL)])
    pltpu.sync_copy(px_h.at[pl.ds(off, SL)], spx.at[pl.ds(off, SL)])
    pltpu.sync_copy(py_h.at[pl.ds(off, SL)], spy.at[pl.ds(off, SL)])
    pltpu.sync_copy(pz_h.at[pl.ds(off, SL)], spz.at[pl.ds(off, SL)])
    pltpu.sync_copy(z_h.at[pl.ds(off, SL)], spzz.at[pl.ds(off, SL)])
    plsc.subcore_barrier()

    ebase = w * ERT

    def echunk(cc, carry):
        r0 = ebase + cc * ECH
        pltpu.sync_copy(col_h.at[pl.ds(r0, ECH)], colv)
        pltpu.sync_copy(row_h.at[pl.ds(r0, ECH)], rowv)
        descs = [
            pltpu.async_copy(spzz.at[colv.at[j]], zg.at[j], sem_g)
            for j in range(ECH)
        ]
        for d in descs:
            d.wait()
        descs = [
            pltpu.async_copy(zg.at[j], sp_t.at[rowv.at[j]], sem_s, add=True)
            for j in range(ECH)
        ]
        for d in descs:
            d.wait()
        return carry

    lax.fori_loop(0, ENC, echunk, 0)

    fbase = w * FRT

    def fchunk(cc, carry):
        r0 = fbase + cc * FCH
        pltpu.sync_copy(fa_h.at[pl.ds(r0, FCH)], fav)
        pltpu.sync_copy(fb_h.at[pl.ds(r0, FCH)], fbv)
        pltpu.sync_copy(fc_h.at[pl.ds(r0, FCH)], fcv)
        descs = []
        for iv, gx, gy, gz in ((fav, gax, gay, gaz),
                               (fbv, gbx, gby, gbz),
                               (fcv, gcx, gcy, gcz)):
            for j in range(FCH):
                descs.append(pltpu.async_copy(spx.at[iv.at[j]], gx.at[j], sem_g))
                descs.append(pltpu.async_copy(spy.at[iv.at[j]], gy.at[j], sem_g))
                descs.append(pltpu.async_copy(spz.at[iv.at[j]], gz.at[j], sem_g))
        for d in descs:
            d.wait()
        for j in range(FCH):
            for i in range(128 // 16):
                sl = pl.ds(i * 16, 16)
                ax = gax[j, sl]
                ay = gay[j, sl]
                az = gaz[j, sl]
                ux = gbx[j, sl] - ax
                uy = gby[j, sl] - ay
                uz = gbz[j, sl] - az
                vx = gcx[j, sl] - ax
                vy = gcy[j, sl] - ay
                vz = gcz[j, sl] - az
                cx = uy * vz - uz * vy
                cy = uz * vx - ux * vz
                cz = ux * vy - uy * vx
                r = _rsqrt_sc(cx * cx + cy * cy + cz * cz)
                vnx[j, sl] = cx * r
                vny[j, sl] = cy * r
                vnz[j, sl] = cz * r
        descs = []
        for iv in (fav, fbv, fcv):
            for j in range(FCH):
                descs.append(
                    pltpu.async_copy(vnx.at[j], sp_nx.at[iv.at[j]], sem_s, add=True))
                descs.append(
                    pltpu.async_copy(vny.at[j], sp_ny.at[iv.at[j]], sem_s, add=True))
                descs.append(
                    pltpu.async_copy(vnz.at[j], sp_nz.at[iv.at[j]], sem_s, add=True))
        for d in descs:
            d.wait()
        return carry

    lax.fori_loop(0, FNC, fchunk, 0)
    plsc.subcore_barrier()
    pltpu.sync_copy(sp_t.at[pl.ds(off, SL)], t_o.at[c, pl.ds(off, SL)])
    pltpu.sync_copy(sp_nx.at[pl.ds(off, SL)], nxo.at[c, pl.ds(off, SL)])
    pltpu.sync_copy(sp_ny.at[pl.ds(off, SL)], nyo.at[c, pl.ds(off, SL)])
    pltpu.sync_copy(sp_nz.at[pl.ds(off, SL)], nzo.at[c, pl.ds(off, SL)])


_a_call = functools.partial(
    pl.kernel,
    out_type=tuple(jax.ShapeDtypeStruct((NC, NPAD), _F32) for _ in range(4)),
    mesh=_MESH,
    scratch_types=[
        pltpu.VMEM((ECH, 128), _I32),
        pltpu.VMEM((ECH, 128), _I32),
        pltpu.VMEM((ECH, 128), _F32),
        pltpu.VMEM((FCH, 128), _I32),
        pltpu.VMEM((FCH, 128), _I32),
        pltpu.VMEM((FCH, 128), _I32),
        pltpu.VMEM((FCH, 128), _F32),
        pltpu.VMEM((FCH, 128), _F32),
        pltpu.VMEM((FCH, 128), _F32),
        pltpu.VMEM((FCH, 128), _F32),
        pltpu.VMEM((FCH, 128), _F32),
        pltpu.VMEM((FCH, 128), _F32),
        pltpu.VMEM((FCH, 128), _F32),
        pltpu.VMEM((FCH, 128), _F32),
        pltpu.VMEM((FCH, 128), _F32),
        pltpu.VMEM((FCH, 128), _F32),
        pltpu.VMEM((FCH, 128), _F32),
        pltpu.VMEM((FCH, 128), _F32),
        pltpu.VMEM((SL,), _F32),
        pltpu.VMEM_SHARED((NPAD,), _F32),
        pltpu.VMEM_SHARED((NPAD,), _F32),
        pltpu.VMEM_SHARED((NPAD,), _F32),
        pltpu.VMEM_SHARED((NPAD,), _F32),
        pltpu.VMEM_SHARED((NPAD,), _F32),
        pltpu.VMEM_SHARED((NPAD,), _F32),
        pltpu.VMEM_SHARED((NPAD,), _F32),
        pltpu.VMEM_SHARED((NPAD,), _F32),
        pltpu.SemaphoreType.DMA,
        pltpu.SemaphoreType.DMA,
    ],
)(_a_body)


def _d0_body(deg_ref, xx, xy, xz, w1, dis_o, z_o):
    deg = deg_ref[0] + deg_ref[1]
    dis = jnp.where(deg > 0, 1.0 / jnp.sqrt(jnp.maximum(deg, 1e-12)), 0.0)
    sv = xx[...] * w1[0] + xy[...] * w1[1] + xz[...] * w1[2]
    dis_o[...] = dis
    z_o[...] = dis * sv


_d0_call = pl.pallas_call(
    _d0_body,
    out_shape=(jax.ShapeDtypeStruct((RB, 128), _F32),
               jax.ShapeDtypeStruct((RB, 128), _F32)),
    in_specs=[
        pl.BlockSpec(memory_space=pltpu.VMEM),
        pl.BlockSpec(memory_space=pltpu.VMEM),
        pl.BlockSpec(memory_space=pltpu.VMEM),
        pl.BlockSpec(memory_space=pltpu.VMEM),
        pl.BlockSpec(memory_space=pltpu.SMEM),
    ],
)


def _u_body(tp, npx, npy, npz, px, py, pz, xx, xy, xz, dx, dy, dz, dis,
            w0, bb, w1n,
            dxo, dyo, dzo, pxo, pyo, pzo, zo):
    nx = npx[0] + npx[1]
    ny = npy[0] + npy[1]
    nz = npz[0] + npz[1]
    nrm = jnp.sqrt(nx * nx + ny * ny + nz * nz)
    inv = 1.0 / jnp.maximum(nrm, 1e-12)
    nx = nx * inv
    ny = ny * inv
    nz = nz * inv
    acc = tp[0] + tp[1]
    dpn = (px[...] * w0[0] + py[...] * w0[1] + pz[...] * w0[2]
           - dis[...] * acc + bb[0])
    ndx = dx[...] + dpn * nx
    ndy = dy[...] + dpn * ny
    ndz = dz[...] + dpn * nz
    npx_ = xx[...] + ndx
    npy_ = xy[...] + ndy
    npz_ = xz[...] + ndz
    dxo[...] = ndx
    dyo[...] = ndy
    dzo[...] = ndz
    pxo[...] = npx_
    pyo[...] = npy_
    pzo[...] = npz_
    zo[...] = dis[...] * (npx_ * w1n[0] + npy_ * w1n[1] + npz_ * w1n[2])


_u_call = pl.pallas_call(
    _u_body,
    out_shape=tuple(jax.ShapeDtypeStruct((RB, 128), _F32) for _ in range(7)),
    in_specs=(
        [pl.BlockSpec(memory_space=pltpu.VMEM) for _ in range(14)]
        + [pl.BlockSpec(memory_space=pltpu.SMEM) for _ in range(3)]
    ),
)


@jax.jit
def kernel(x, edge_index, faces, W1, b1, W2, b2, W3, b3):
    epad = jnp.full((EPAD - E,), N, _I32)
    col2 = jnp.concatenate([edge_index[1].astype(_I32), epad]).reshape(ER, 128)
    row2 = jnp.concatenate([edge_index[0].astype(_I32), epad]).reshape(ER, 128)
    fpad = jnp.full((FPAD - FC,), N, _I32)
    fa2 = jnp.concatenate([faces[:, 0].astype(_I32), fpad]).reshape(FR, 128)
    fb2 = jnp.concatenate([faces[:, 1].astype(_I32), fpad]).reshape(FR, 128)
    fc2 = jnp.concatenate([faces[:, 2].astype(_I32), fpad]).reshape(FR, 128)

    xx = jnp.pad(x[:, 0], (0, NPAD - N)).reshape(RB, 128)
    xy = jnp.pad(x[:, 1], (0, NPAD - N)).reshape(RB, 128)
    xz = jnp.pad(x[:, 2], (0, NPAD - N)).reshape(RB, 128)

    degp = _p_call(row2).reshape(NC, RB, 128)
    dis, z = _d0_call(degp, xx, xy, xz, W1[1, :, 0])

    px, py, pz = xx, xy, xz
    dx = jnp.zeros((RB, 128), _F32)
    dy = jnp.zeros((RB, 128), _F32)
    dz = jnp.zeros((RB, 128), _F32)
    params = [(W1, b1), (W2, b2), (W3, b3)]
    for i in range(3):
        tp, npx, npy, npz = _a_call(
            col2, row2, fa2, fb2, fc2,
            px.reshape(NPAD), py.reshape(NPAD), pz.reshape(NPAD),
            z.reshape(NPAD))
        w0 = params[i][0][0, :, 0]
        bb = params[i][1]
        w1n = params[(i + 1) % 3][0][1, :, 0]
        dx, dy, dz, px, py, pz, z = _u_call(
            tp.reshape(NC, RB, 128), npx.reshape(NC, RB, 128),
            npy.reshape(NC, RB, 128), npz.reshape(NC, RB, 128),
            px, py, pz, xx, xy, xz, dx, dy, dz, dis,
            w0, bb, w1n)

    dpos = jnp.stack(
        [dx.reshape(NPAD)[:N], dy.reshape(NPAD)[:N], dz.reshape(NPAD)[:N]],
        axis=1)
    return (dpos, dpos, dpos)


# speedup vs baseline: 57.9778x; 57.9778x over previous
import functools

import jax
import jax.numpy as jnp
from jax import lax
from jax.experimental import pallas as pl
from jax.experimental.pallas import tpu as pltpu
from jax.experimental.pallas import tpu_sc as plsc

N = 100000
E = 1600000
FC = 200000

NC = 2
NS = 16
NW = NC * NS

NPAD = 100096
SL = NPAD // NS
RB = NPAD // 128

ECH = 16
EPAD = 1638400
ER = EPAD // 128
ERT = ER // NW
ENC = ERT // ECH

FCH = 2
FPAD = 204800
FR = FPAD // 128
FRT = FR // NW
FNC = FRT // FCH

_MESH = plsc.VectorSubcoreMesh(core_axis_name="c", subcore_axis_name="s")
_F32 = jnp.float32
_I32 = jnp.int32


def _fill(ref, n, value):
    def body(i, carry):
        ref[pl.ds(i * 16, 16)] = jnp.full((16,), value, _F32)
        return carry
    lax.fori_loop(0, n // 16, body, 0)


def _rsqrt_sc(nn):
    i = lax.bitcast_convert_type(nn, _I32)
    i = jnp.int32(0x5F3759DF) - lax.shift_right_logical(i, 1)
    y = lax.bitcast_convert_type(i, _F32)
    for _ in range(3):
        y = y * (1.5 - 0.5 * nn * y * y)
    return y


def _p_body(row_h, deg_o, rowv, onesv, zerov, sp_deg, sem_s):
    c = lax.axis_index("c")
    s = lax.axis_index("s")
    w = c * NS + s
    off = s * SL
    _fill(zerov, SL, 0.0)
    _fill(onesv, 128, 1.0)
    pltpu.sync_copy(zerov, sp_deg.at[pl.ds(off, SL)])
    plsc.subcore_barrier()

    ebase = w * ENC

    def chunk(cc, carry):
        pltpu.sync_copy(row_h.at[ebase + cc], rowv)
        descs = [
            pltpu.async_copy(onesv, sp_deg.at[rowv.at[j]], sem_s, add=True)
            for j in range(ECH)
        ]
        for d in descs:
            d.wait()
        return carry

    lax.fori_loop(0, ENC, chunk, 0)
    plsc.subcore_barrier()
    pltpu.sync_copy(sp_deg.at[pl.ds(off, SL)], zerov)
    pltpu.sync_copy(zerov, deg_o.at[pl.ds(c * NPAD + off, SL)])


_p_call = functools.partial(
    pl.kernel,
    out_type=jax.ShapeDtypeStruct((NC * NPAD,), _F32),
    mesh=_MESH,
    scratch_types=[
        pltpu.VMEM((ECH, 128), _I32),
        pltpu.VMEM((128,), _F32),
        pltpu.VMEM((SL,), _F32),
        pltpu.VMEM_SHARED((NPAD,), _F32),
        pltpu.SemaphoreType.DMA,
    ],
)(_p_body)


def _a_body(col_h, row_h, fa_h, fb_h, fc_h, px_h, py_h, pz_h, z_h,
            t_o, nxo, nyo, nzo,
            colv, rowv, zg, fav, fbv, fcv,
            gax, gay, gaz, gbx, gby, gbz, gcx, gcy, gcz,
            vnx, vny, vnz, zerov,
            spx, spy, spz, spzz, sp_t, sp_nx, sp_ny, sp_nz,
            sem_g, sem_s):
    c = lax.axis_index("c")
    s = lax.axis_index("s")
    w = c * NS + s
    off = s * SL

    _fill(zerov, SL, 0.0)
    pltpu.sync_copy(zerov, sp_t.at[pl.ds(off, SL)])
    pltpu.sync_copy(zerov, sp_nx.at[pl.ds(off, SL)])
    pltpu.sync_copy(zerov, sp_ny.at[pl.ds(off, SL)])
    pltpu.sync_copy(zerov, sp_nz.at[pl.ds(off, SL)])
    for hsrc, sdst in ((px_h, spx), (py_h, spy), (pz_h, spz), (z_h, spzz)):
        pltpu.sync_copy(hsrc.at[pl.ds(off, SL)], zerov)
        pltpu.sync_copy(zerov, sdst.at[pl.ds(off, SL)])
    plsc.subcore_barrier()

    ebase = w * ENC

    def echunk(cc, carry):
        pltpu.sync_copy(col_h.at[ebase + cc], colv)
        pltpu.sync_copy(row_h.at[ebase + cc], rowv)
        descs = [
            pltpu.async_copy(spzz.at[colv.at[j]], zg.at[j], sem_g)
            for j in range(ECH)
        ]
        for d in descs:
            d.wait()
        descs = [
            pltpu.async_copy(zg.at[j], sp_t.at[rowv.at[j]], sem_s, add=True)
            for j in range(ECH)
        ]
        for d in descs:
            d.wait()
        return carry

    lax.fori_loop(0, ENC, echunk, 0)

    fbase = w * FNC

    def fchunk(cc, carry):
        pltpu.sync_copy(fa_h.at[fbase + cc], fav)
        pltpu.sync_copy(fb_h.at[fbase + cc], fbv)
        pltpu.sync_copy(fc_h.at[fbase + cc], fcv)
        descs = []
        for iv, gx, gy, gz in ((fav, gax, gay, gaz),
                               (fbv, gbx, gby, gbz),
                               (fcv, gcx, gcy, gcz)):
            for j in range(FCH):
                descs.append(pltpu.async_copy(spx.at[iv.at[j]], gx.at[j], sem_g))
                descs.append(pltpu.async_copy(spy.at[iv.at[j]], gy.at[j], sem_g))
                descs.append(pltpu.async_copy(spz.at[iv.at[j]], gz.at[j], sem_g))
        for d in descs:
            d.wait()
        for j in range(FCH):
            for i in range(128 // 16):
                sl = pl.ds(i * 16, 16)
                ax = gax[j, sl]
                ay = gay[j, sl]
                az = gaz[j, sl]
                ux = gbx[j, sl] - ax
                uy = gby[j, sl] - ay
                uz = gbz[j, sl] - az
                vx = gcx[j, sl] - ax
                vy = gcy[j, sl] - ay
                vz = gcz[j, sl] - az
                cx = uy * vz - uz * vy
                cy = uz * vx - ux * vz
                cz = ux * vy - uy * vx
                r = _rsqrt_sc(cx * cx + cy * cy + cz * cz)
                vnx[j, sl] = cx * r
                vny[j, sl] = cy * r
                vnz[j, sl] = cz * r
        descs = []
        for iv in (fav, fbv, fcv):
            for j in range(FCH):
                descs.append(
                    pltpu.async_copy(vnx.at[j], sp_nx.at[iv.at[j]], sem_s, add=True))
                descs.append(
                    pltpu.async_copy(vny.at[j], sp_ny.at[iv.at[j]], sem_s, add=True))
                descs.append(
                    pltpu.async_copy(vnz.at[j], sp_nz.at[iv.at[j]], sem_s, add=True))
        for d in descs:
            d.wait()
        return carry

    lax.fori_loop(0, FNC, fchunk, 0)
    plsc.subcore_barrier()
    o = c * NPAD + off
    for ssrc, hdst in ((sp_t, t_o), (sp_nx, nxo), (sp_ny, nyo), (sp_nz, nzo)):
        pltpu.sync_copy(ssrc.at[pl.ds(off, SL)], zerov)
        pltpu.sync_copy(zerov, hdst.at[pl.ds(o, SL)])


_a_call = functools.partial(
    pl.kernel,
    out_type=tuple(jax.ShapeDtypeStruct((NC * NPAD,), _F32) for _ in range(4)),
    mesh=_MESH,
    scratch_types=[
        pltpu.VMEM((ECH, 128), _I32),
        pltpu.VMEM((ECH, 128), _I32),
        pltpu.VMEM((ECH, 128), _F32),
        pltpu.VMEM((FCH, 128), _I32),
        pltpu.VMEM((FCH, 128), _I32),
        pltpu.VMEM((FCH, 128), _I32),
        pltpu.VMEM((FCH, 128), _F32),
        pltpu.VMEM((FCH, 128), _F32),
        pltpu.VMEM((FCH, 128), _F32),
        pltpu.VMEM((FCH, 128), _F32),
        pltpu.VMEM((FCH, 128), _F32),
        pltpu.VMEM((FCH, 128), _F32),
        pltpu.VMEM((FCH, 128), _F32),
        pltpu.VMEM((FCH, 128), _F32),
        pltpu.VMEM((FCH, 128), _F32),
        pltpu.VMEM((FCH, 128), _F32),
        pltpu.VMEM((FCH, 128), _F32),
        pltpu.VMEM((FCH, 128), _F32),
        pltpu.VMEM((SL,), _F32),
        pltpu.VMEM_SHARED((NPAD,), _F32),
        pltpu.VMEM_SHARED((NPAD,), _F32),
        pltpu.VMEM_SHARED((NPAD,), _F32),
        pltpu.VMEM_SHARED((NPAD,), _F32),
        pltpu.VMEM_SHARED((NPAD,), _F32),
        pltpu.VMEM_SHARED((NPAD,), _F32),
        pltpu.VMEM_SHARED((NPAD,), _F32),
        pltpu.VMEM_SHARED((NPAD,), _F32),
        pltpu.SemaphoreType.DMA,
        pltpu.SemaphoreType.DMA,
    ],
)(_a_body)


def _d0_body(deg_ref, xx, xy, xz, w1, dis_o, z_o):
    deg = deg_ref[0] + deg_ref[1]
    dis = jnp.where(deg > 0, 1.0 / jnp.sqrt(jnp.maximum(deg, 1e-12)), 0.0)
    sv = xx[...] * w1[0] + xy[...] * w1[1] + xz[...] * w1[2]
    dis_o[...] = dis
    z_o[...] = dis * sv


_d0_call = pl.pallas_call(
    _d0_body,
    out_shape=(jax.ShapeDtypeStruct((RB, 128), _F32),
               jax.ShapeDtypeStruct((RB, 128), _F32)),
    in_specs=[
        pl.BlockSpec(memory_space=pltpu.VMEM),
        pl.BlockSpec(memory_space=pltpu.VMEM),
        pl.BlockSpec(memory_space=pltpu.VMEM),
        pl.BlockSpec(memory_space=pltpu.VMEM),
        pl.BlockSpec(memory_space=pltpu.SMEM),
    ],
)


def _u_body(tp, npx, npy, npz, px, py, pz, xx, xy, xz, dx, dy, dz, dis,
            w0, bb, w1n,
            dxo, dyo, dzo, pxo, pyo, pzo, zo):
    nx = npx[0] + npx[1]
    ny = npy[0] + npy[1]
    nz = npz[0] + npz[1]
    nrm = jnp.sqrt(nx * nx + ny * ny + nz * nz)
    inv = 1.0 / jnp.maximum(nrm, 1e-12)
    nx = nx * inv
    ny = ny * inv
    nz = nz * inv
    acc = tp[0] + tp[1]
    dpn = (px[...] * w0[0] + py[...] * w0[1] + pz[...] * w0[2]
           - dis[...] * acc + bb[0])
    ndx = dx[...] + dpn * nx
    ndy = dy[...] + dpn * ny
    ndz = dz[...] + dpn * nz
    npx_ = xx[...] + ndx
    npy_ = xy[...] + ndy
    npz_ = xz[...] + ndz
    dxo[...] = ndx
    dyo[...] = ndy
    dzo[...] = ndz
    pxo[...] = npx_
    pyo[...] = npy_
    pzo[...] = npz_
    zo[...] = dis[...] * (npx_ * w1n[0] + npy_ * w1n[1] + npz_ * w1n[2])


_u_call = pl.pallas_call(
    _u_body,
    out_shape=tuple(jax.ShapeDtypeStruct((RB, 128), _F32) for _ in range(7)),
    in_specs=(
        [pl.BlockSpec(memory_space=pltpu.VMEM) for _ in range(14)]
        + [pl.BlockSpec(memory_space=pltpu.SMEM) for _ in range(3)]
    ),
)


@jax.jit
def kernel(x, edge_index, faces, W1, b1, W2, b2, W3, b3):
    epad = jnp.full((EPAD - E,), N, _I32)
    col2 = jnp.concatenate([edge_index[1].astype(_I32), epad]).reshape(
        NW * ENC, ECH, 128)
    row2 = jnp.concatenate([edge_index[0].astype(_I32), epad]).reshape(
        NW * ENC, ECH, 128)
    fpad = jnp.full((FPAD - FC,), N, _I32)
    fa2 = jnp.concatenate([faces[:, 0].astype(_I32), fpad]).reshape(
        NW * FNC, FCH, 128)
    fb2 = jnp.concatenate([faces[:, 1].astype(_I32), fpad]).reshape(
        NW * FNC, FCH, 128)
    fc2 = jnp.concatenate([faces[:, 2].astype(_I32), fpad]).reshape(
        NW * FNC, FCH, 128)

    xx = jnp.pad(x[:, 0], (0, NPAD - N)).reshape(RB, 128)
    xy = jnp.pad(x[:, 1], (0, NPAD - N)).reshape(RB, 128)
    xz = jnp.pad(x[:, 2], (0, NPAD - N)).reshape(RB, 128)

    degp = _p_call(row2).reshape(NC, RB, 128)
    dis, z = _d0_call(degp, xx, xy, xz, W1[1, :, 0])

    px, py, pz = xx, xy, xz
    dx = jnp.zeros((RB, 128), _F32)
    dy = jnp.zeros((RB, 128), _F32)
    dz = jnp.zeros((RB, 128), _F32)
    params = [(W1, b1), (W2, b2), (W3, b3)]
    for i in range(3):
        tp, npx, npy, npz = _a_call(
            col2, row2, fa2, fb2, fc2,
            px.reshape(NPAD), py.reshape(NPAD), pz.reshape(NPAD),
            z.reshape(NPAD))
        w0 = params[i][0][0, :, 0]
        bb = params[i][1]
        w1n = params[(i + 1) % 3][0][1, :, 0]
        dx, dy, dz, px, py, pz, z = _u_call(
            tp.reshape(NC, RB, 128), npx.reshape(NC, RB, 128),
            npy.reshape(NC, RB, 128), npz.reshape(NC, RB, 128),
            px, py, pz, xx, xy, xz, dx, dy, dz, dis,
            w0, bb, w1n)

    dpos = jnp.stack(
        [dx.reshape(NPAD)[:N], dy.reshape(NPAD)[:N], dz.reshape(NPAD)[:N]],
        axis=1)
    return (dpos, dpos, dpos)


# speedup vs baseline: 122.0227x; 2.1046x over previous
import functools

import jax
import jax.numpy as jnp
from jax import lax
from jax.experimental import pallas as pl
from jax.experimental.pallas import tpu as pltpu
from jax.experimental.pallas import tpu_sc as plsc

N = 100000
E = 1600000
FC = 200000

NC = 2
NS = 16
NW = NC * NS

NPAD = 100096
SL = NPAD // NS
RB = NPAD // 128

CE = 3968
ENCT = 13
EPAD = NW * ENCT * CE
ETOT = NW * ENCT

CF = 640
FNCT = 10
FPAD = NW * FNCT * CF
FTOT = NW * FNCT

_MESH = plsc.VectorSubcoreMesh(core_axis_name="c", subcore_axis_name="s")
_F32 = jnp.float32
_I32 = jnp.int32


def _fill(ref, n, value):
    def body(i, carry):
        ref[pl.ds(i * 16, 16)] = jnp.full((16,), value, _F32)
        return carry
    lax.fori_loop(0, n // 16, body, 0)


def _rsqrt_sc(nn):
    i = lax.bitcast_convert_type(nn, _I32)
    i = jnp.int32(0x5F3759DF) - lax.shift_right_logical(i, 1)
    y = lax.bitcast_convert_type(i, _F32)
    for _ in range(3):
        y = y * (1.5 - 0.5 * nn * y * y)
    return y


def _p_body(row_h, deg_o, rv0, rv1, rv2, onesv, zerov, sp_deg, sem_s):
    c = lax.axis_index("c")
    s = lax.axis_index("s")
    w = c * NS + s
    off = s * SL
    rvs = (rv0, rv1, rv2)
    _fill(zerov, SL, 0.0)
    _fill(onesv, CE, 1.0)
    pltpu.sync_copy(zerov, sp_deg.at[pl.ds(off, SL)])
    plsc.subcore_barrier()

    base = w * ENCT

    def load(st, g):
        pltpu.sync_copy(row_h.at[pl.ds(g * CE, CE)], rvs[st])

    def fire(st):
        pltpu.async_copy(onesv, sp_deg.at[rvs[st]], sem_s, add=True)

    def drain(st):
        pltpu.make_async_copy(onesv, sp_deg.at[rvs[st]], sem_s).wait()

    load(0, base)
    fire(0)
    load(1, base + 1)

    def body(b, carry):
        for k in range(3):
            cc = 3 * b + 1 + k
            cur = (1 + k) % 3
            nxt = (2 + k) % 3
            prv = k % 3
            fire(cur)
            load(nxt, lax.rem(base + cc + 1, ETOT))
            drain(prv)
        return carry

    lax.fori_loop(0, (ENCT - 1) // 3, body, 0)
    drain(0)
    plsc.subcore_barrier()
    pltpu.sync_copy(sp_deg.at[pl.ds(off, SL)], zerov)
    pltpu.sync_copy(zerov, deg_o.at[pl.ds(c * NPAD + off, SL)])


_p_call = functools.partial(
    pl.kernel,
    out_type=jax.ShapeDtypeStruct((NC * NPAD,), _F32),
    mesh=_MESH,
    scratch_types=[
        pltpu.VMEM((CE,), _I32),
        pltpu.VMEM((CE,), _I32),
        pltpu.VMEM((CE,), _I32),
        pltpu.VMEM((CE,), _F32),
        pltpu.VMEM((SL,), _F32),
        pltpu.VMEM_SHARED((NPAD,), _F32),
        pltpu.SemaphoreType.DMA,
    ],
)(_p_body)


def _a_body(col_h, row_h, fa_h, fb_h, fc_h, px_h, py_h, pz_h, z_h,
            t_o, nxo, nyo, nzo,
            cv0, cv1, cv2, rv0, rv1, rv2, zg0, zg1, zg2,
            fa0, fa1, fa2, fb0, fb1, fb2, fc0, fc1, fc2,
            gx0, gx1, gx2, gy0, gy1, gy2, gz0, gz1, gz2,
            hx0, hx1, hx2, hy0, hy1, hy2, hz0, hz1, hz2,
            kx0, kx1, kx2, ky0, ky1, ky2, kz0, kz1, kz2,
            vx0, vx1, vx2, vy0, vy1, vy2, vz0, vz1, vz2,
            zerov,
            spx, spy, spz, spzz, sp_t, sp_nx, sp_ny, sp_nz,
            sem_g, sem_s):
    c = lax.axis_index("c")
    s = lax.axis_index("s")
    w = c * NS + s
    off = s * SL

    cvs = (cv0, cv1, cv2)
    rvs = (rv0, rv1, rv2)
    zgs = (zg0, zg1, zg2)
    fas = (fa0, fa1, fa2)
    fbs = (fb0, fb1, fb2)
    fcs = (fc0, fc1, fc2)
    gxs = (gx0, gx1, gx2)
    gys = (gy0, gy1, gy2)
    gzs = (gz0, gz1, gz2)
    hxs = (hx0, hx1, hx2)
    hys = (hy0, hy1, hy2)
    hzs = (hz0, hz1, hz2)
    kxs = (kx0, kx1, kx2)
    kys = (ky0, ky1, ky2)
    kzs = (kz0, kz1, kz2)
    vxs = (vx0, vx1, vx2)
    vys = (vy0, vy1, vy2)
    vzs = (vz0, vz1, vz2)

    _fill(zerov, SL, 0.0)
    pltpu.sync_copy(zerov, sp_t.at[pl.ds(off, SL)])
    pltpu.sync_copy(zerov, sp_nx.at[pl.ds(off, SL)])
    pltpu.sync_copy(zerov, sp_ny.at[pl.ds(off, SL)])
    pltpu.sync_copy(zerov, sp_nz.at[pl.ds(off, SL)])
    for hsrc, sdst in ((px_h, spx), (py_h, spy), (pz_h, spz), (z_h, spzz)):
        pltpu.sync_copy(hsrc.at[pl.ds(off, SL)], zerov)
        pltpu.sync_copy(zerov, sdst.at[pl.ds(off, SL)])
    plsc.subcore_barrier()

    ebase = w * ENCT

    def eload(st, g):
        pltpu.sync_copy(col_h.at[pl.ds(g * CE, CE)], cvs[st])
        pltpu.sync_copy(row_h.at[pl.ds(g * CE, CE)], rvs[st])

    def eg_fire(st):
        pltpu.async_copy(spzz.at[cvs[st]], zgs[st], sem_g)

    def eg_drain(st):
        pltpu.make_async_copy(spzz.at[cvs[st]], zgs[st], sem_g).wait()

    def es_fire(st):
        pltpu.async_copy(zgs[st], sp_t.at[rvs[st]], sem_s, add=True)

    def es_drain(st):
        pltpu.make_async_copy(zgs[st], sp_t.at[rvs[st]], sem_s).wait()

    eload(0, ebase)
    eg_fire(0)
    eload(1, ebase + 1)
    eg_drain(0)
    eg_fire(1)
    es_fire(0)

    def ebody(b, carry):
        for k in range(3):
            cc = 3 * b + 1 + k
            cur = (1 + k) % 3
            nxt = (2 + k) % 3
            prv = k % 3
            eload(nxt, lax.rem(ebase + cc + 1, ETOT))
            eg_drain(cur)
            eg_fire(nxt)
            es_fire(cur)
            es_drain(prv)
        return carry

    lax.fori_loop(0, (ENCT - 1) // 3, ebody, 0)
    eg_drain(1)
    es_drain(0)

    fbase = w * FNCT

    def fload(st, g):
        pltpu.sync_copy(fa_h.at[pl.ds(g * CF, CF)], fas[st])
        pltpu.sync_copy(fb_h.at[pl.ds(g * CF, CF)], fbs[st])
        pltpu.sync_copy(fc_h.at[pl.ds(g * CF, CF)], fcs[st])

    def fg_fire(st):
        pltpu.async_copy(spx.at[fas[st]], gxs[st], sem_g)
        pltpu.async_copy(spy.at[fas[st]], gys[st], sem_g)
        pltpu.async_copy(spz.at[fas[st]], gzs[st], sem_g)
        pltpu.async_copy(spx.at[fbs[st]], hxs[st], sem_g)
        pltpu.async_copy(spy.at[fbs[st]], hys[st], sem_g)
        pltpu.async_copy(spz.at[fbs[st]], hzs[st], sem_g)
        pltpu.async_copy(spx.at[fcs[st]], kxs[st], sem_g)
        pltpu.async_copy(spy.at[fcs[st]], kys[st], sem_g)
        pltpu.async_copy(spz.at[fcs[st]], kzs[st], sem_g)

    def fg_drain(st):
        pltpu.make_async_copy(spx.at[fas[st]], gxs[st], sem_g).wait()
        pltpu.make_async_copy(spy.at[fas[st]], gys[st], sem_g).wait()
        pltpu.make_async_copy(spz.at[fas[st]], gzs[st], sem_g).wait()
        pltpu.make_async_copy(spx.at[fbs[st]], hxs[st], sem_g).wait()
        pltpu.make_async_copy(spy.at[fbs[st]], hys[st], sem_g).wait()
        pltpu.make_async_copy(spz.at[fbs[st]], hzs[st], sem_g).wait()
        pltpu.make_async_copy(spx.at[fcs[st]], kxs[st], sem_g).wait()
        pltpu.make_async_copy(spy.at[fcs[st]], kys[st], sem_g).wait()
        pltpu.make_async_copy(spz.at[fcs[st]], kzs[st], sem_g).wait()

    def fs_fire(st):
        pltpu.async_copy(vxs[st], sp_nx.at[fas[st]], sem_s, add=True)
        pltpu.async_copy(vys[st], sp_ny.at[fas[st]], sem_s, add=True)
        pltpu.async_copy(vzs[st], sp_nz.at[fas[st]], sem_s, add=True)
        pltpu.async_copy(vxs[st], sp_nx.at[fbs[st]], sem_s, add=True)
        pltpu.async_copy(vys[st], sp_ny.at[fbs[st]], sem_s, add=True)
        pltpu.async_copy(vzs[st], sp_nz.at[fbs[st]], sem_s, add=True)
        pltpu.async_copy(vxs[st], sp_nx.at[fcs[st]], sem_s, add=True)
        pltpu.async_copy(vys[st], sp_ny.at[fcs[st]], sem_s, add=True)
        pltpu.async_copy(vzs[st], sp_nz.at[fcs[st]], sem_s, add=True)

    def fs_drain(st):
        pltpu.make_async_copy(vxs[st], sp_nx.at[fas[st]], sem_s).wait()
        pltpu.make_async_copy(vys[st], sp_ny.at[fas[st]], sem_s).wait()
        pltpu.make_async_copy(vzs[st], sp_nz.at[fas[st]], sem_s).wait()
        pltpu.make_async_copy(vxs[st], sp_nx.at[fbs[st]], sem_s).wait()
        pltpu.make_async_copy(vys[st], sp_ny.at[fbs[st]], sem_s).wait()
        pltpu.make_async_copy(vzs[st], sp_nz.at[fbs[st]], sem_s).wait()
        pltpu.make_async_copy(vxs[st], sp_nx.at[fcs[st]], sem_s).wait()
        pltpu.make_async_copy(vys[st], sp_ny.at[fcs[st]], sem_s).wait()
        pltpu.make_async_copy(vzs[st], sp_nz.at[fcs[st]], sem_s).wait()

    def fcompute(st):
        gx, gy, gz = gxs[st], gys[st], gzs[st]
        hx, hy, hz = hxs[st], hys[st], hzs[st]
        kx, ky, kz = kxs[st], kys[st], kzs[st]
        vx, vy, vz = vxs[st], vys[st], vzs[st]

        def grp(i, carry):
            sl = pl.ds(i * 16, 16)
            ax = gx[sl]
            ay = gy[sl]
            az = gz[sl]
            ux = hx[sl] - ax
            uy = hy[sl] - ay
            uz = hz[sl] - az
            wx = kx[sl] - ax
            wy = ky[sl] - ay
            wz = kz[sl] - az
            cx = uy * wz - uz * wy
            cy = uz * wx - ux * wz
            cz = ux * wy - uy * wx
            r = _rsqrt_sc(cx * cx + cy * cy + cz * cz)
            vx[sl] = cx * r
            vy[sl] = cy * r
            vz[sl] = cz * r
            return carry

        lax.fori_loop(0, CF // 16, grp, 0)

    fload(0, fbase)
    fg_fire(0)
    fload(1, fbase + 1)
    fg_drain(0)
    fg_fire(1)
    fcompute(0)
    fs_fire(0)

    def fbody(b, carry):
        for k in range(3):
            cc = 3 * b + 1 + k
            cur = (1 + k) % 3
            nxt = (2 + k) % 3
            prv = k % 3
            fload(nxt, lax.rem(fbase + cc + 1, FTOT))
            fg_drain(cur)
            fg_fire(nxt)
            fcompute(cur)
            fs_fire(cur)
            fs_drain(prv)
        return carry

    lax.fori_loop(0, (FNCT - 1) // 3, fbody, 0)
    fg_drain(1)
    fs_drain(0)

    plsc.subcore_barrier()
    o = c * NPAD + off
    for ssrc, hdst in ((sp_t, t_o), (sp_nx, nxo), (sp_ny, nyo), (sp_nz, nzo)):
        pltpu.sync_copy(ssrc.at[pl.ds(off, SL)], zerov)
        pltpu.sync_copy(zerov, hdst.at[pl.ds(o, SL)])


_a_scratch = (
    [pltpu.VMEM((CE,), _I32) for _ in range(6)]
    + [pltpu.VMEM((CE,), _F32) for _ in range(3)]
    + [pltpu.VMEM((CF,), _I32) for _ in range(9)]
    + [pltpu.VMEM((CF,), _F32) for _ in range(36)]
    + [pltpu.VMEM((SL,), _F32)]
    + [pltpu.VMEM_SHARED((NPAD,), _F32) for _ in range(8)]
    + [pltpu.SemaphoreType.DMA, pltpu.SemaphoreType.DMA]
)

_a_call = functools.partial(
    pl.kernel,
    out_type=tuple(jax.ShapeDtypeStruct((NC * NPAD,), _F32) for _ in range(4)),
    mesh=_MESH,
    scratch_types=_a_scratch,
)(_a_body)


def _d0_body(deg_ref, xx, xy, xz, w1, dis_o, z_o):
    deg = deg_ref[0] + deg_ref[1]
    dis = jnp.where(deg > 0, 1.0 / jnp.sqrt(jnp.maximum(deg, 1e-12)), 0.0)
    sv = xx[...] * w1[0] + xy[...] * w1[1] + xz[...] * w1[2]
    dis_o[...] = dis
    z_o[...] = dis * sv


_d0_call = pl.pallas_call(
    _d0_body,
    out_shape=(jax.ShapeDtypeStruct((RB, 128), _F32),
               jax.ShapeDtypeStruct((RB, 128), _F32)),
    in_specs=[
        pl.BlockSpec(memory_space=pltpu.VMEM),
        pl.BlockSpec(memory_space=pltpu.VMEM),
        pl.BlockSpec(memory_space=pltpu.VMEM),
        pl.BlockSpec(memory_space=pltpu.VMEM),
        pl.BlockSpec(memory_space=pltpu.SMEM),
    ],
)


def _u_body(tp, npx, npy, npz, px, py, pz, xx, xy, xz, dx, dy, dz, dis,
            w0, bb, w1n,
            dxo, dyo, dzo, pxo, pyo, pzo, zo):
    nx = npx[0] + npx[1]
    ny = npy[0] + npy[1]
    nz = npz[0] + npz[1]
    nrm = jnp.sqrt(nx * nx + ny * ny + nz * nz)
    inv = 1.0 / jnp.maximum(nrm, 1e-12)
    nx = nx * inv
    ny = ny * inv
    nz = nz * inv
    acc = tp[0] + tp[1]
    dpn = (px[...] * w0[0] + py[...] * w0[1] + pz[...] * w0[2]
           - dis[...] * acc + bb[0])
    ndx = dx[...] + dpn * nx
    ndy = dy[...] + dpn * ny
    ndz = dz[...] + dpn * nz
    npx_ = xx[...] + ndx
    npy_ = xy[...] + ndy
    npz_ = xz[...] + ndz
    dxo[...] = ndx
    dyo[...] = ndy
    dzo[...] = ndz
    pxo[...] = npx_
    pyo[...] = npy_
    pzo[...] = npz_
    zo[...] = dis[...] * (npx_ * w1n[0] + npy_ * w1n[1] + npz_ * w1n[2])


_u_call = pl.pallas_call(
    _u_body,
    out_shape=tuple(jax.ShapeDtypeStruct((RB, 128), _F32) for _ in range(7)),
    in_specs=(
        [pl.BlockSpec(memory_space=pltpu.VMEM) for _ in range(14)]
        + [pl.BlockSpec(memory_space=pltpu.SMEM) for _ in range(3)]
    ),
)


@jax.jit
def kernel(x, edge_index, faces, W1, b1, W2, b2, W3, b3):
    epadi = (jnp.arange(EPAD - E, dtype=_I32) % (NPAD - N)) + N
    col1 = jnp.concatenate([edge_index[1].astype(_I32), epadi])
    row1 = jnp.concatenate([edge_index[0].astype(_I32), epadi])
    fpadi = (jnp.arange(FPAD - FC, dtype=_I32) % (NPAD - N)) + N
    fa1 = jnp.concatenate([faces[:, 0].astype(_I32), fpadi])
    fb1 = jnp.concatenate([faces[:, 1].astype(_I32), fpadi])
    fc1 = jnp.concatenate([faces[:, 2].astype(_I32), fpadi])

    xx = jnp.pad(x[:, 0], (0, NPAD - N)).reshape(RB, 128)
    xy = jnp.pad(x[:, 1], (0, NPAD - N)).reshape(RB, 128)
    xz = jnp.pad(x[:, 2], (0, NPAD - N)).reshape(RB, 128)

    degp = _p_call(row1).reshape(NC, RB, 128)
    dis, z = _d0_call(degp, xx, xy, xz, W1[1, :, 0])

    px, py, pz = xx, xy, xz
    dx = jnp.zeros((RB, 128), _F32)
    dy = jnp.zeros((RB, 128), _F32)
    dz = jnp.zeros((RB, 128), _F32)
    params = [(W1, b1), (W2, b2), (W3, b3)]
    for i in range(3):
        tp, npx, npy, npz = _a_call(
            col1, row1, fa1, fb1, fc1,
            px.reshape(NPAD), py.reshape(NPAD), pz.reshape(NPAD),
            z.reshape(NPAD))
        w0 = params[i][0][0, :, 0]
        bb = params[i][1]
        w1n = params[(i + 1) % 3][0][1, :, 0]
        dx, dy, dz, px, py, pz, z = _u_call(
            tp.reshape(NC, RB, 128), npx.reshape(NC, RB, 128),
            npy.reshape(NC, RB, 128), npz.reshape(NC, RB, 128),
            px, py, pz, xx, xy, xz, dx, dy, dz, dis,
            w0, bb, w1n)

    dpos = jnp.stack(
        [dx.reshape(NPAD)[:N], dy.reshape(NPAD)[:N], dz.reshape(NPAD)[:N]],
        axis=1)
    return (dpos, dpos, dpos)


# speedup vs baseline: 123.8359x; 1.0149x over previous
import functools

import jax
import jax.numpy as jnp
from jax import lax
from jax.experimental import pallas as pl
from jax.experimental.pallas import tpu as pltpu
from jax.experimental.pallas import tpu_sc as plsc

N = 100000
E = 1600000
FC = 200000

NC = 2
NS = 16
NW = NC * NS

NPAD = 100096
SL = NPAD // NS
RB = NPAD // 128

CE = 3968
ENCT = 13
EPAD = NW * ENCT * CE
ETOT = NW * ENCT

CF = 640
FNCT = 10
FPAD = NW * FNCT * CF
FTOT = NW * FNCT

_MESH = plsc.VectorSubcoreMesh(core_axis_name="c", subcore_axis_name="s")
_F32 = jnp.float32
_I32 = jnp.int32


def _fill(ref, n, value):
    def body(i, carry):
        ref[pl.ds(i * 16, 16)] = jnp.full((16,), value, _F32)
        return carry
    lax.fori_loop(0, n // 16, body, 0)


def _rsqrt_sc(nn):
    i = lax.bitcast_convert_type(nn, _I32)
    i = jnp.int32(0x5F3759DF) - lax.shift_right_logical(i, 1)
    y = lax.bitcast_convert_type(i, _F32)
    for _ in range(3):
        y = y * (1.5 - 0.5 * nn * y * y)
    return y


CE2 = 2048
EFULL2 = E // CE2
ETAIL2 = E - EFULL2 * CE2
UCE2 = EFULL2 // NW
EEXTRA2 = EFULL2 - UCE2 * NW
EPADB = (EPAD - E) // NW

CF2 = 1920
FFULL2 = FC // CF2
FTAIL2 = FC - FFULL2 * CF2
UCF2 = FFULL2 // NW
FEXTRA2 = FFULL2 - UCF2 * NW
FPADW = 12
FPADB = (FPAD - FC) // FPADW


def _deint_clamp(src2d, dsts, n):
    def body(i, carry):
        sl = pl.ds(i * 16, 16)
        for j, d in enumerate(dsts):
            v = src2d[j, sl]
            d[sl] = jnp.minimum(jnp.maximum(v, 0), NPAD - 1)
        return carry
    lax.fori_loop(0, n // 16, body, 0)


def _padfill(ref, n, base):
    def body(i, carry):
        idx = lax.iota(_I32, 16) + (base + i * 16)
        ref[pl.ds(i * 16, 16)] = jnp.int32(N) + lax.rem(idx,
                                                        jnp.int32(NPAD - N))
        return carry
    lax.fori_loop(0, n // 16, body, 0)


def _r_body(ei_h, ft_h, row_o, col_o, fa_o, fb_o, fc_o,
            ev0, ev1, ev2, rw0, rw1, rw2, cl0, cl1, cl2,
            fv0, fv1, fv2, pa0, pa1, pa2, pb0, pb1, pb2, pc0, pc1, pc2,
            etailv, trw, tcl, ftailv, tfa, tfb, tfc, padb, fpadb, sem):
    c = lax.axis_index("c")
    s = lax.axis_index("s")
    w = c * NS + s
    evs = (ev0, ev1, ev2)
    rws = (rw0, rw1, rw2)
    cls = (cl0, cl1, cl2)
    fvs = (fv0, fv1, fv2)
    pas = (pa0, pa1, pa2)
    pbs = (pb0, pb1, pb2)
    pcs = (pc0, pc1, pc2)

    def eproc(st, g):
        pltpu.sync_copy(ei_h.at[pl.ds(0, 2), pl.ds(g * CE2, CE2)], evs[st])
        _deint_clamp(evs[st], (rws[st], cls[st]), CE2)
        pltpu.async_copy(rws[st], row_o.at[pl.ds(g * CE2, CE2)], sem)
        pltpu.async_copy(cls[st], col_o.at[pl.ds(g * CE2, CE2)], sem)

    def edrain(st):
        pltpu.make_async_copy(rws[st], row_o.at[pl.ds(0, CE2)], sem).wait()
        pltpu.make_async_copy(cls[st], col_o.at[pl.ds(0, CE2)], sem).wait()

    eproc(0, w)
    eproc(1, NW + w)
    eproc(2, 2 * NW + w)

    def ebody(b, carry):
        for k in range(3):
            cc = 3 * b + 3 + k
            edrain(k)
            eproc(k, cc * NW + w)
        return carry

    lax.fori_loop(0, (UCE2 - 3) // 3, ebody, 0)
    edrain(0)
    edrain(1)
    edrain(2)

    @pl.when(w < EEXTRA2)
    def _():
        eproc(0, UCE2 * NW + w)
        edrain(0)

    @pl.when(w == EEXTRA2)
    def _():
        pltpu.sync_copy(ei_h.at[pl.ds(0, 2), pl.ds(EFULL2 * CE2, ETAIL2)],
                        etailv)
        _deint_clamp(etailv, (trw, tcl), ETAIL2)
        pltpu.async_copy(trw, row_o.at[pl.ds(EFULL2 * CE2, ETAIL2)], sem)
        pltpu.async_copy(tcl, col_o.at[pl.ds(EFULL2 * CE2, ETAIL2)], sem)
        pltpu.make_async_copy(trw, row_o.at[pl.ds(0, ETAIL2)], sem).wait()
        pltpu.make_async_copy(tcl, col_o.at[pl.ds(0, ETAIL2)], sem).wait()

    _padfill(padb, EPADB, w * EPADB)
    pltpu.sync_copy(padb, row_o.at[pl.ds(E + w * EPADB, EPADB)])
    pltpu.sync_copy(padb, col_o.at[pl.ds(E + w * EPADB, EPADB)])

    def fproc(st, g):
        pltpu.sync_copy(ft_h.at[pl.ds(0, 3), pl.ds(g * CF2, CF2)], fvs[st])
        _deint_clamp(fvs[st], (pas[st], pbs[st], pcs[st]), CF2)
        pltpu.async_copy(pas[st], fa_o.at[pl.ds(g * CF2, CF2)], sem)
        pltpu.async_copy(pbs[st], fb_o.at[pl.ds(g * CF2, CF2)], sem)
        pltpu.async_copy(pcs[st], fc_o.at[pl.ds(g * CF2, CF2)], sem)

    def fdrain(st):
        pltpu.make_async_copy(pas[st], fa_o.at[pl.ds(0, CF2)], sem).wait()
        pltpu.make_async_copy(pbs[st], fb_o.at[pl.ds(0, CF2)], sem).wait()
        pltpu.make_async_copy(pcs[st], fc_o.at[pl.ds(0, CF2)], sem).wait()

    fproc(0, w)
    fproc(1, NW + w)
    fproc(2, 2 * NW + w)
    fdrain(0)
    fdrain(1)
    fdrain(2)

    @pl.when(w < FEXTRA2)
    def _():
        fproc(0, UCF2 * NW + w)
        fdrain(0)

    @pl.when(w == FEXTRA2)
    def _():
        pltpu.sync_copy(ft_h.at[pl.ds(0, 3), pl.ds(FFULL2 * CF2, FTAIL2)],
                        ftailv)
        _deint_clamp(ftailv, (tfa, tfb, tfc), FTAIL2)
        pltpu.async_copy(tfa, fa_o.at[pl.ds(FFULL2 * CF2, FTAIL2)], sem)
        pltpu.async_copy(tfb, fb_o.at[pl.ds(FFULL2 * CF2, FTAIL2)], sem)
        pltpu.async_copy(tfc, fc_o.at[pl.ds(FFULL2 * CF2, FTAIL2)], sem)
        pltpu.make_async_copy(tfa, fa_o.at[pl.ds(0, FTAIL2)], sem).wait()
        pltpu.make_async_copy(tfb, fb_o.at[pl.ds(0, FTAIL2)], sem).wait()
        pltpu.make_async_copy(tfc, fc_o.at[pl.ds(0, FTAIL2)], sem).wait()

    @pl.when(w < FPADW)
    def _():
        _padfill(fpadb, FPADB, w * FPADB)
        pltpu.sync_copy(fpadb, fa_o.at[pl.ds(FC + w * FPADB, FPADB)])
        pltpu.sync_copy(fpadb, fb_o.at[pl.ds(FC + w * FPADB, FPADB)])
        pltpu.sync_copy(fpadb, fc_o.at[pl.ds(FC + w * FPADB, FPADB)])


_r_call = functools.partial(
    pl.kernel,
    out_type=(jax.ShapeDtypeStruct((EPAD,), _I32),
              jax.ShapeDtypeStruct((EPAD,), _I32),
              jax.ShapeDtypeStruct((FPAD,), _I32),
              jax.ShapeDtypeStruct((FPAD,), _I32),
              jax.ShapeDtypeStruct((FPAD,), _I32)),
    mesh=_MESH,
    scratch_types=(
        [pltpu.VMEM((2, CE2), _I32) for _ in range(3)]
        + [pltpu.VMEM((CE2,), _I32) for _ in range(6)]
        + [pltpu.VMEM((3, CF2), _I32) for _ in range(3)]
        + [pltpu.VMEM((CF2,), _I32) for _ in range(9)]
        + [pltpu.VMEM((2, ETAIL2), _I32),
           pltpu.VMEM((ETAIL2,), _I32),
           pltpu.VMEM((ETAIL2,), _I32),
           pltpu.VMEM((3, FTAIL2), _I32),
           pltpu.VMEM((FTAIL2,), _I32),
           pltpu.VMEM((FTAIL2,), _I32),
           pltpu.VMEM((FTAIL2,), _I32),
           pltpu.VMEM((EPADB,), _I32),
           pltpu.VMEM((FPADB,), _I32),
           pltpu.SemaphoreType.DMA]
    ),
)(_r_body)


def _p_body(row_h, deg_o, rv0, rv1, rv2, onesv, zerov, sp_deg, sem_s):
    c = lax.axis_index("c")
    s = lax.axis_index("s")
    w = c * NS + s
    off = s * SL
    rvs = (rv0, rv1, rv2)
    _fill(zerov, SL, 0.0)
    _fill(onesv, CE, 1.0)
    pltpu.sync_copy(zerov, sp_deg.at[pl.ds(off, SL)])
    plsc.subcore_barrier()

    base = w * ENCT

    def load(st, g):
        pltpu.sync_copy(row_h.at[pl.ds(g * CE, CE)], rvs[st])

    def fire(st):
        pltpu.async_copy(onesv, sp_deg.at[rvs[st]], sem_s, add=True)

    def drain(st):
        pltpu.make_async_copy(onesv, sp_deg.at[rvs[st]], sem_s).wait()

    load(0, base)
    fire(0)
    load(1, base + 1)

    def body(b, carry):
        for k in range(3):
            cc = 3 * b + 1 + k
            cur = (1 + k) % 3
            nxt = (2 + k) % 3
            prv = k % 3
            fire(cur)
            load(nxt, lax.rem(base + cc + 1, ETOT))
            drain(prv)
        return carry

    lax.fori_loop(0, (ENCT - 1) // 3, body, 0)
    drain(0)
    plsc.subcore_barrier()
    pltpu.sync_copy(sp_deg.at[pl.ds(off, SL)], zerov)
    pltpu.sync_copy(zerov, deg_o.at[pl.ds(c * NPAD + off, SL)])


_p_call = functools.partial(
    pl.kernel,
    out_type=jax.ShapeDtypeStruct((NC * NPAD,), _F32),
    mesh=_MESH,
    scratch_types=[
        pltpu.VMEM((CE,), _I32),
        pltpu.VMEM((CE,), _I32),
        pltpu.VMEM((CE,), _I32),
        pltpu.VMEM((CE,), _F32),
        pltpu.VMEM((SL,), _F32),
        pltpu.VMEM_SHARED((NPAD,), _F32),
        pltpu.SemaphoreType.DMA,
    ],
)(_p_body)


def _a_body(col_h, row_h, fa_h, fb_h, fc_h, px_h, py_h, pz_h, z_h,
            t_o, nxo, nyo, nzo,
            cv0, cv1, cv2, rv0, rv1, rv2, zg0, zg1, zg2,
            fa0, fa1, fa2, fb0, fb1, fb2, fc0, fc1, fc2,
            gx0, gx1, gx2, gy0, gy1, gy2, gz0, gz1, gz2,
            hx0, hx1, hx2, hy0, hy1, hy2, hz0, hz1, hz2,
            kx0, kx1, kx2, ky0, ky1, ky2, kz0, kz1, kz2,
            vx0, vx1, vx2, vy0, vy1, vy2, vz0, vz1, vz2,
            zerov,
            spx, spy, spz, spzz, sp_t, sp_nx, sp_ny, sp_nz,
            sem_g, sem_s):
    c = lax.axis_index("c")
    s = lax.axis_index("s")
    w = c * NS + s
    off = s * SL

    cvs = (cv0, cv1, cv2)
    rvs = (rv0, rv1, rv2)
    zgs = (zg0, zg1, zg2)
    fas = (fa0, fa1, fa2)
    fbs = (fb0, fb1, fb2)
    fcs = (fc0, fc1, fc2)
    gxs = (gx0, gx1, gx2)
    gys = (gy0, gy1, gy2)
    gzs = (gz0, gz1, gz2)
    hxs = (hx0, hx1, hx2)
    hys = (hy0, hy1, hy2)
    hzs = (hz0, hz1, hz2)
    kxs = (kx0, kx1, kx2)
    kys = (ky0, ky1, ky2)
    kzs = (kz0, kz1, kz2)
    vxs = (vx0, vx1, vx2)
    vys = (vy0, vy1, vy2)
    vzs = (vz0, vz1, vz2)

    _fill(zerov, SL, 0.0)
    pltpu.sync_copy(zerov, sp_t.at[pl.ds(off, SL)])
    pltpu.sync_copy(zerov, sp_nx.at[pl.ds(off, SL)])
    pltpu.sync_copy(zerov, sp_ny.at[pl.ds(off, SL)])
    pltpu.sync_copy(zerov, sp_nz.at[pl.ds(off, SL)])
    for hsrc, sdst in ((px_h, spx), (py_h, spy), (pz_h, spz), (z_h, spzz)):
        pltpu.sync_copy(hsrc.at[pl.ds(off, SL)], zerov)
        pltpu.sync_copy(zerov, sdst.at[pl.ds(off, SL)])
    plsc.subcore_barrier()

    ebase = w * ENCT

    def eload(st, g):
        pltpu.sync_copy(col_h.at[pl.ds(g * CE, CE)], cvs[st])
        pltpu.sync_copy(row_h.at[pl.ds(g * CE, CE)], rvs[st])

    def eg_fire(st):
        pltpu.async_copy(spzz.at[cvs[st]], zgs[st], sem_g)

    def eg_drain(st):
        pltpu.make_async_copy(spzz.at[cvs[st]], zgs[st], sem_g).wait()

    def es_fire(st):
        pltpu.async_copy(zgs[st], sp_t.at[rvs[st]], sem_s, add=True)

    def es_drain(st):
        pltpu.make_async_copy(zgs[st], sp_t.at[rvs[st]], sem_s).wait()

    eload(0, ebase)
    eg_fire(0)
    eload(1, ebase + 1)
    eg_drain(0)
    eg_fire(1)
    es_fire(0)

    def ebody(b, carry):
        for k in range(3):
            cc = 3 * b + 1 + k
            cur = (1 + k) % 3
            nxt = (2 + k) % 3
            prv = k % 3
            eload(nxt, lax.rem(ebase + cc + 1, ETOT))
            eg_drain(cur)
            eg_fire(nxt)
            es_fire(cur)
            es_drain(prv)
        return carry

    lax.fori_loop(0, (ENCT - 1) // 3, ebody, 0)
    eg_drain(1)
    es_drain(0)

    fbase = w * FNCT

    def fload(st, g):
        pltpu.sync_copy(fa_h.at[pl.ds(g * CF, CF)], fas[st])
        pltpu.sync_copy(fb_h.at[pl.ds(g * CF, CF)], fbs[st])
        pltpu.sync_copy(fc_h.at[pl.ds(g * CF, CF)], fcs[st])

    def fg_fire(st):
        pltpu.async_copy(spx.at[fas[st]], gxs[st], sem_g)
        pltpu.async_copy(spy.at[fas[st]], gys[st], sem_g)
        pltpu.async_copy(spz.at[fas[st]], gzs[st], sem_g)
        pltpu.async_copy(spx.at[fbs[st]], hxs[st], sem_g)
        pltpu.async_copy(spy.at[fbs[st]], hys[st], sem_g)
        pltpu.async_copy(spz.at[fbs[st]], hzs[st], sem_g)
        pltpu.async_copy(spx.at[fcs[st]], kxs[st], sem_g)
        pltpu.async_copy(spy.at[fcs[st]], kys[st], sem_g)
        pltpu.async_copy(spz.at[fcs[st]], kzs[st], sem_g)

    def fg_drain(st):
        pltpu.make_async_copy(spx.at[fas[st]], gxs[st], sem_g).wait()
        pltpu.make_async_copy(spy.at[fas[st]], gys[st], sem_g).wait()
        pltpu.make_async_copy(spz.at[fas[st]], gzs[st], sem_g).wait()
        pltpu.make_async_copy(spx.at[fbs[st]], hxs[st], sem_g).wait()
        pltpu.make_async_copy(spy.at[fbs[st]], hys[st], sem_g).wait()
        pltpu.make_async_copy(spz.at[fbs[st]], hzs[st], sem_g).wait()
        pltpu.make_async_copy(spx.at[fcs[st]], kxs[st], sem_g).wait()
        pltpu.make_async_copy(spy.at[fcs[st]], kys[st], sem_g).wait()
        pltpu.make_async_copy(spz.at[fcs[st]], kzs[st], sem_g).wait()

    def fs_fire(st):
        pltpu.async_copy(vxs[st], sp_nx.at[fas[st]], sem_s, add=True)
        pltpu.async_copy(vys[st], sp_ny.at[fas[st]], sem_s, add=True)
        pltpu.async_copy(vzs[st], sp_nz.at[fas[st]], sem_s, add=True)
        pltpu.async_copy(vxs[st], sp_nx.at[fbs[st]], sem_s, add=True)
        pltpu.async_copy(vys[st], sp_ny.at[fbs[st]], sem_s, add=True)
        pltpu.async_copy(vzs[st], sp_nz.at[fbs[st]], sem_s, add=True)
        pltpu.async_copy(vxs[st], sp_nx.at[fcs[st]], sem_s, add=True)
        pltpu.async_copy(vys[st], sp_ny.at[fcs[st]], sem_s, add=True)
        pltpu.async_copy(vzs[st], sp_nz.at[fcs[st]], sem_s, add=True)

    def fs_drain(st):
        pltpu.make_async_copy(vxs[st], sp_nx.at[fas[st]], sem_s).wait()
        pltpu.make_async_copy(vys[st], sp_ny.at[fas[st]], sem_s).wait()
        pltpu.make_async_copy(vzs[st], sp_nz.at[fas[st]], sem_s).wait()
        pltpu.make_async_copy(vxs[st], sp_nx.at[fbs[st]], sem_s).wait()
        pltpu.make_async_copy(vys[st], sp_ny.at[fbs[st]], sem_s).wait()
        pltpu.make_async_copy(vzs[st], sp_nz.at[fbs[st]], sem_s).wait()
        pltpu.make_async_copy(vxs[st], sp_nx.at[fcs[st]], sem_s).wait()
        pltpu.make_async_copy(vys[st], sp_ny.at[fcs[st]], sem_s).wait()
        pltpu.make_async_copy(vzs[st], sp_nz.at[fcs[st]], sem_s).wait()

    def fcompute(st):
        gx, gy, gz = gxs[st], gys[st], gzs[st]
        hx, hy, hz = hxs[st], hys[st], hzs[st]
        kx, ky, kz = kxs[st], kys[st], kzs[st]
        vx, vy, vz = vxs[st], vys[st], vzs[st]

        def grp(i, carry):
            sl = pl.ds(i * 16, 16)
            ax = gx[sl]
            ay = gy[sl]
            az = gz[sl]
            ux = hx[sl] - ax
            uy = hy[sl] - ay
            uz = hz[sl] - az
            wx = kx[sl] - ax
            wy = ky[sl] - ay
            wz = kz[sl] - az
            cx = uy * wz - uz * wy
            cy = uz * wx - ux * wz
            cz = ux * wy - uy * wx
            r = _rsqrt_sc(cx * cx + cy * cy + cz * cz)
            vx[sl] = cx * r
            vy[sl] = cy * r
            vz[sl] = cz * r
            return carry

        lax.fori_loop(0, CF // 16, grp, 0)

    fload(0, fbase)
    fg_fire(0)
    fload(1, fbase + 1)
    fg_drain(0)
    fg_fire(1)
    fcompute(0)
    fs_fire(0)

    def fbody(b, carry):
        for k in range(3):
            cc = 3 * b + 1 + k
            cur = (1 + k) % 3
            nxt = (2 + k) % 3
            prv = k % 3
            fload(nxt, lax.rem(fbase + cc + 1, FTOT))
            fg_drain(cur)
            fg_fire(nxt)
            fcompute(cur)
            fs_fire(cur)
            fs_drain(prv)
        return carry

    lax.fori_loop(0, (FNCT - 1) // 3, fbody, 0)
    fg_drain(1)
    fs_drain(0)

    plsc.subcore_barrier()
    o = c * NPAD + off
    for ssrc, hdst in ((sp_t, t_o), (sp_nx, nxo), (sp_ny, nyo), (sp_nz, nzo)):
        pltpu.sync_copy(ssrc.at[pl.ds(off, SL)], zerov)
        pltpu.sync_copy(zerov, hdst.at[pl.ds(o, SL)])


_a_scratch = (
    [pltpu.VMEM((CE,), _I32) for _ in range(6)]
    + [pltpu.VMEM((CE,), _F32) for _ in range(3)]
    + [pltpu.VMEM((CF,), _I32) for _ in range(9)]
    + [pltpu.VMEM((CF,), _F32) for _ in range(36)]
    + [pltpu.VMEM((SL,), _F32)]
    + [pltpu.VMEM_SHARED((NPAD,), _F32) for _ in range(8)]
    + [pltpu.SemaphoreType.DMA, pltpu.SemaphoreType.DMA]
)

_a_call = functools.partial(
    pl.kernel,
    out_type=tuple(jax.ShapeDtypeStruct((NC * NPAD,), _F32) for _ in range(4)),
    mesh=_MESH,
    scratch_types=_a_scratch,
)(_a_body)


def _d0_body(deg_ref, xx, xy, xz, w1, dis_o, z_o):
    deg = deg_ref[0] + deg_ref[1]
    dis = jnp.where(deg > 0, 1.0 / jnp.sqrt(jnp.maximum(deg, 1e-12)), 0.0)
    sv = xx[...] * w1[0] + xy[...] * w1[1] + xz[...] * w1[2]
    dis_o[...] = dis
    z_o[...] = dis * sv


_d0_call = pl.pallas_call(
    _d0_body,
    out_shape=(jax.ShapeDtypeStruct((RB, 128), _F32),
               jax.ShapeDtypeStruct((RB, 128), _F32)),
    in_specs=[
        pl.BlockSpec(memory_space=pltpu.VMEM),
        pl.BlockSpec(memory_space=pltpu.VMEM),
        pl.BlockSpec(memory_space=pltpu.VMEM),
        pl.BlockSpec(memory_space=pltpu.VMEM),
        pl.BlockSpec(memory_space=pltpu.SMEM),
    ],
)


def _u_body(tp, npx, npy, npz, px, py, pz, xx, xy, xz, dx, dy, dz, dis,
            w0, bb, w1n,
            dxo, dyo, dzo, pxo, pyo, pzo, zo):
    nx = npx[0] + npx[1]
    ny = npy[0] + npy[1]
    nz = npz[0] + npz[1]
    nrm = jnp.sqrt(nx * nx + ny * ny + nz * nz)
    inv = 1.0 / jnp.maximum(nrm, 1e-12)
    nx = nx * inv
    ny = ny * inv
    nz = nz * inv
    acc = tp[0] + tp[1]
    dpn = (px[...] * w0[0] + py[...] * w0[1] + pz[...] * w0[2]
           - dis[...] * acc + bb[0])
    ndx = dx[...] + dpn * nx
    ndy = dy[...] + dpn * ny
    ndz = dz[...] + dpn * nz
    npx_ = xx[...] + ndx
    npy_ = xy[...] + ndy
    npz_ = xz[...] + ndz
    dxo[...] = ndx
    dyo[...] = ndy
    dzo[...] = ndz
    pxo[...] = npx_
    pyo[...] = npy_
    pzo[...] = npz_
    zo[...] = dis[...] * (npx_ * w1n[0] + npy_ * w1n[1] + npz_ * w1n[2])


_u_call = pl.pallas_call(
    _u_body,
    out_shape=tuple(jax.ShapeDtypeStruct((RB, 128), _F32) for _ in range(7)),
    in_specs=(
        [pl.BlockSpec(memory_space=pltpu.VMEM) for _ in range(14)]
        + [pl.BlockSpec(memory_space=pltpu.SMEM) for _ in range(3)]
    ),
)


@jax.jit
def kernel(x, edge_index, faces, W1, b1, W2, b2, W3, b3):
    ei = edge_index.astype(_I32)
    ft = faces.astype(_I32).T
    row1, col1, fa1, fb1, fc1 = _r_call(ei, ft)

    xx = jnp.pad(x[:, 0], (0, NPAD - N)).reshape(RB, 128)
    xy = jnp.pad(x[:, 1], (0, NPAD - N)).reshape(RB, 128)
    xz = jnp.pad(x[:, 2], (0, NPAD - N)).reshape(RB, 128)

    degp = _p_call(row1).reshape(NC, RB, 128)
    dis, z = _d0_call(degp, xx, xy, xz, W1[1, :, 0])

    px, py, pz = xx, xy, xz
    dx = jnp.zeros((RB, 128), _F32)
    dy = jnp.zeros((RB, 128), _F32)
    dz = jnp.zeros((RB, 128), _F32)
    params = [(W1, b1), (W2, b2), (W3, b3)]
    for i in range(3):
        tp, npx, npy, npz = _a_call(
            col1, row1, fa1, fb1, fc1,
            px.reshape(NPAD), py.reshape(NPAD), pz.reshape(NPAD),
            z.reshape(NPAD))
        w0 = params[i][0][0, :, 0]
        bb = params[i][1]
        w1n = params[(i + 1) % 3][0][1, :, 0]
        dx, dy, dz, px, py, pz, z = _u_call(
            tp.reshape(NC, RB, 128), npx.reshape(NC, RB, 128),
            npy.reshape(NC, RB, 128), npz.reshape(NC, RB, 128),
            px, py, pz, xx, xy, xz, dx, dy, dz, dis,
            w0, bb, w1n)

    dpos = jnp.stack(
        [dx.reshape(NPAD)[:N], dy.reshape(NPAD)[:N], dz.reshape(NPAD)[:N]],
        axis=1)
    return (dpos, dpos, dpos)


# speedup vs baseline: 129.1758x; 1.0431x over previous
import functools

import jax
import jax.numpy as jnp
from jax import lax
from jax.experimental import pallas as pl
from jax.experimental.pallas import tpu as pltpu
from jax.experimental.pallas import tpu_sc as plsc

N = 100000
E = 1600000
FC = 200000

NC = 2
NS = 16
NW = NC * NS

NPAD = 100096
SL = NPAD // NS
RB = NPAD // 128

CE = 2048
ENCT = 25
EFULL = E // CE

CF = 640
FNCT = 10
FFULL = FC // CF
FTPAD = 320

_MESH = plsc.VectorSubcoreMesh(core_axis_name="c", subcore_axis_name="s")
_F32 = jnp.float32
_I32 = jnp.int32


def _fill(ref, n, value):
    def body(i, carry):
        ref[pl.ds(i * 16, 16)] = jnp.full((16,), value, _F32)
        return carry
    lax.fori_loop(0, n // 16, body, 0)


def _rsqrt_sc(nn):
    i = lax.bitcast_convert_type(nn, _I32)
    i = jnp.int32(0x5F3759DF) - lax.shift_right_logical(i, 1)
    y = lax.bitcast_convert_type(i, _F32)
    for _ in range(3):
        y = y * (1.5 - 0.5 * nn * y * y)
    return y


def _deint(src2d, dsts, n):
    def body(i, carry):
        sl = pl.ds(i * 16, 16)
        for j, d in enumerate(dsts):
            if d is not None:
                v = src2d[j, sl]
                d[sl] = jnp.minimum(jnp.maximum(v, 0), NPAD - 1)
        return carry
    lax.fori_loop(0, n // 16, body, 0)


def _deint_sel(src2d, dsts, n, lo, hi):
    def body(i, carry):
        sl = pl.ds(i * 16, 16)
        pos = lax.iota(_I32, 16) + i * 16
        keep = jnp.logical_and(pos >= lo, pos < hi)
        dummy = jnp.int32(N) + lax.rem(pos, jnp.int32(NPAD - N))
        for j, d in enumerate(dsts):
            if d is not None:
                v = src2d[j, sl]
                v = jnp.minimum(jnp.maximum(v, 0), NPAD - 1)
                d[sl] = jnp.where(keep, v, dummy)
        return carry
    lax.fori_loop(0, n // 16, body, 0)


def _edge_last_load(ei_h, w, ev, dsts):
    g = 24 * NW + w
    offs = jnp.minimum(g * CE, E - CE)
    pltpu.sync_copy(ei_h.at[pl.ds(0, 2), pl.ds(offs, CE)], ev)
    lo = jnp.where(w <= 12, 0, CE - 512)
    hi = jnp.where(w <= 13, CE, 0)
    _deint_sel(ev, dsts, CE, lo, hi)


def _face_last_load(ft_h, w, fv, dsts):
    g = 9 * NW + w
    offs = jnp.minimum(g, FFULL) * CF
    pltpu.sync_copy(ft_h.at[pl.ds(0, 3), pl.ds(offs, CF)], fv)
    hi = jnp.where(w < 24, CF, jnp.where(w == 24, FC - FFULL * CF, 0))
    _deint_sel(fv, dsts, CF, 0, hi)


def _p_body(ei_h, deg_o, ev0, ev1, ev2, rw0, rw1, rw2, onesv, zerov,
            sp_deg, sem_s):
    c = lax.axis_index("c")
    s = lax.axis_index("s")
    w = c * NS + s
    off = s * SL
    evs = (ev0, ev1, ev2)
    rws = (rw0, rw1, rw2)
    _fill(zerov, SL, 0.0)
    _fill(onesv, CE, 1.0)
    pltpu.sync_copy(zerov, sp_deg.at[pl.ds(off, SL)])
    plsc.subcore_barrier()

    def load(st, cc):
        g = cc * NW + w
        pltpu.sync_copy(ei_h.at[pl.ds(0, 2), pl.ds(g * CE, CE)], evs[st])
        _deint(evs[st], (rws[st], None), CE)

    def fire(st):
        pltpu.async_copy(onesv, sp_deg.at[rws[st]], sem_s, add=True)

    def drain(st):
        pltpu.make_async_copy(onesv, sp_deg.at[rws[st]], sem_s).wait()

    load(0, 0)
    fire(0)
    load(1, 1)
    fire(1)
    load(2, 2)
    drain(0)

    def body(b, carry):
        for k in range(3):
            cc = 3 * b + 2 + k
            cur = (2 + k) % 3
            nxt = k % 3
            prv = (1 + k) % 3
            fire(cur)
            load(nxt, cc + 1)
            drain(prv)
        return carry

    lax.fori_loop(0, 7, body, 0)
    fire(2)
    _edge_last_load(ei_h, w, evs[0], (rws[0], None))
    drain(1)
    fire(0)
    drain(2)
    drain(0)

    plsc.subcore_barrier()
    pltpu.sync_copy(sp_deg.at[pl.ds(off, SL)], zerov)
    pltpu.sync_copy(zerov, deg_o.at[pl.ds(c * NPAD + off, SL)])


_p_call = functools.partial(
    pl.kernel,
    out_type=jax.ShapeDtypeStruct((NC * NPAD,), _F32),
    mesh=_MESH,
    scratch_types=[
        pltpu.VMEM((2, CE), _I32),
        pltpu.VMEM((2, CE), _I32),
        pltpu.VMEM((2, CE), _I32),
        pltpu.VMEM((CE,), _I32),
        pltpu.VMEM((CE,), _I32),
        pltpu.VMEM((CE,), _I32),
        pltpu.VMEM((CE,), _F32),
        pltpu.VMEM((SL,), _F32),
        pltpu.VMEM_SHARED((NPAD,), _F32),
        pltpu.SemaphoreType.DMA,
    ],
)(_p_body)


def _a_body(ei_h, ft_h, px_h, py_h, pz_h, z_h,
            t_o, nxo, nyo, nzo,
            ev0, ev1, ev2, rw0, rw1, rw2, cl0, cl1, cl2, zg0, zg1, zg2,
            fv0, fv1, fv2, pa0, pa1, pa2, pb0, pb1, pb2, pc0, pc1, pc2,
            gx0, gx1, gx2, gy0, gy1, gy2, gz0, gz1, gz2,
            hx0, hx1, hx2, hy0, hy1, hy2, hz0, hz1, hz2,
            kx0, kx1, kx2, ky0, ky1, ky2, kz0, kz1, kz2,
            vx0, vx1, vx2, vy0, vy1, vy2, vz0, vz1, vz2,
            zerov,
            spx, spy, spz, spzz, sp_t, sp_nx, sp_ny, sp_nz,
            sem_g, sem_s):
    c = lax.axis_index("c")
    s = lax.axis_index("s")
    w = c * NS + s
    off = s * SL

    evs = (ev0, ev1, ev2)
    rws = (rw0, rw1, rw2)
    cls = (cl0, cl1, cl2)
    zgs = (zg0, zg1, zg2)
    fvs = (fv0, fv1, fv2)
    pas = (pa0, pa1, pa2)
    pbs = (pb0, pb1, pb2)
    pcs = (pc0, pc1, pc2)
    gxs = (gx0, gx1, gx2)
    gys = (gy0, gy1, gy2)
    gzs = (gz0, gz1, gz2)
    hxs = (hx0, hx1, hx2)
    hys = (hy0, hy1, hy2)
    hzs = (hz0, hz1, hz2)
    kxs = (kx0, kx1, kx2)
    kys = (ky0, ky1, ky2)
    kzs = (kz0, kz1, kz2)
    vxs = (vx0, vx1, vx2)
    vys = (vy0, vy1, vy2)
    vzs = (vz0, vz1, vz2)

    _fill(zerov, SL, 0.0)
    pltpu.sync_copy(zerov, sp_t.at[pl.ds(off, SL)])
    pltpu.sync_copy(zerov, sp_nx.at[pl.ds(off, SL)])
    pltpu.sync_copy(zerov, sp_ny.at[pl.ds(off, SL)])
    pltpu.sync_copy(zerov, sp_nz.at[pl.ds(off, SL)])
    for hsrc, sdst in ((px_h, spx), (py_h, spy), (pz_h, spz), (z_h, spzz)):
        pltpu.sync_copy(hsrc.at[pl.ds(off, SL)], zerov)
        pltpu.sync_copy(zerov, sdst.at[pl.ds(off, SL)])
    plsc.subcore_barrier()


    def eload(st, cc):
        g = cc * NW + w
        pltpu.sync_copy(ei_h.at[pl.ds(0, 2), pl.ds(g * CE, CE)], evs[st])
        _deint(evs[st], (rws[st], cls[st]), CE)

    def eg_fire(st):
        pltpu.async_copy(spzz.at[cls[st]], zgs[st], sem_g)

    def eg_drain(st):
        pltpu.make_async_copy(spzz.at[cls[st]], zgs[st], sem_g).wait()

    def es_fire(st):
        pltpu.async_copy(zgs[st], sp_t.at[rws[st]], sem_s, add=True)

    def es_drain(st):
        pltpu.make_async_copy(zgs[st], sp_t.at[rws[st]], sem_s).wait()

    eload(0, 0)
    eg_fire(0)
    eload(1, 1)
    eg_drain(0)
    eg_fire(1)
    es_fire(0)
    eload(2, 2)
    eg_drain(1)
    eg_fire(2)
    es_fire(1)
    es_drain(0)

    def ebody(b, carry):
        for k in range(3):
            cc = 3 * b + 2 + k
            cur = (2 + k) % 3
            nxt = k % 3
            prv = (1 + k) % 3
            eload(nxt, cc + 1)
            eg_drain(cur)
            eg_fire(nxt)
            es_fire(cur)
            es_drain(prv)
        return carry

    lax.fori_loop(0, 7, ebody, 0)
    _edge_last_load(ei_h, w, evs[0], (rws[0], cls[0]))
    eg_drain(2)
    eg_fire(0)
    es_fire(2)
    es_drain(1)
    eg_drain(0)
    es_fire(0)
    es_drain(2)
    es_drain(0)


    def fload(st, cc):
        g = cc * NW + w
        pltpu.sync_copy(ft_h.at[pl.ds(0, 3), pl.ds(g * CF, CF)], fvs[st])
        _deint(fvs[st], (pas[st], pbs[st], pcs[st]), CF)

    def fg_fire(st):
        pltpu.async_copy(spx.at[pas[st]], gxs[st], sem_g)
        pltpu.async_copy(spy.at[pas[st]], gys[st], sem_g)
        pltpu.async_copy(spz.at[pas[st]], gzs[st], sem_g)
        pltpu.async_copy(spx.at[pbs[st]], hxs[st], sem_g)
        pltpu.async_copy(spy.at[pbs[st]], hys[st], sem_g)
        pltpu.async_copy(spz.at[pbs[st]], hzs[st], sem_g)
        pltpu.async_copy(spx.at[pcs[st]], kxs[st], sem_g)
        pltpu.async_copy(spy.at[pcs[st]], kys[st], sem_g)
        pltpu.async_copy(spz.at[pcs[st]], kzs[st], sem_g)

    def fg_drain(st):
        pltpu.make_async_copy(spx.at[pas[st]], gxs[st], sem_g).wait()
        pltpu.make_async_copy(spy.at[pas[st]], gys[st], sem_g).wait()
        pltpu.make_async_copy(spz.at[pas[st]], gzs[st], sem_g).wait()
        pltpu.make_async_copy(spx.at[pbs[st]], hxs[st], sem_g).wait()
        pltpu.make_async_copy(spy.at[pbs[st]], hys[st], sem_g).wait()
        pltpu.make_async_copy(spz.at[pbs[st]], hzs[st], sem_g).wait()
        pltpu.make_async_copy(spx.at[pcs[st]], kxs[st], sem_g).wait()
        pltpu.make_async_copy(spy.at[pcs[st]], kys[st], sem_g).wait()
        pltpu.make_async_copy(spz.at[pcs[st]], kzs[st], sem_g).wait()

    def fs_fire(st):
        pltpu.async_copy(vxs[st], sp_nx.at[pas[st]], sem_s, add=True)
        pltpu.async_copy(vys[st], sp_ny.at[pas[st]], sem_s, add=True)
        pltpu.async_copy(vzs[st], sp_nz.at[pas[st]], sem_s, add=True)
        pltpu.async_copy(vxs[st], sp_nx.at[pbs[st]], sem_s, add=True)
        pltpu.async_copy(vys[st], sp_ny.at[pbs[st]], sem_s, add=True)
        pltpu.async_copy(vzs[st], sp_nz.at[pbs[st]], sem_s, add=True)
        pltpu.async_copy(vxs[st], sp_nx.at[pcs[st]], sem_s, add=True)
        pltpu.async_copy(vys[st], sp_ny.at[pcs[st]], sem_s, add=True)
        pltpu.async_copy(vzs[st], sp_nz.at[pcs[st]], sem_s, add=True)

    def fs_drain(st):
        pltpu.make_async_copy(vxs[st], sp_nx.at[pas[st]], sem_s).wait()
        pltpu.make_async_copy(vys[st], sp_ny.at[pas[st]], sem_s).wait()
        pltpu.make_async_copy(vzs[st], sp_nz.at[pas[st]], sem_s).wait()
        pltpu.make_async_copy(vxs[st], sp_nx.at[pbs[st]], sem_s).wait()
        pltpu.make_async_copy(vys[st], sp_ny.at[pbs[st]], sem_s).wait()
        pltpu.make_async_copy(vzs[st], sp_nz.at[pbs[st]], sem_s).wait()
        pltpu.make_async_copy(vxs[st], sp_nx.at[pcs[st]], sem_s).wait()
        pltpu.make_async_copy(vys[st], sp_ny.at[pcs[st]], sem_s).wait()
        pltpu.make_async_copy(vzs[st], sp_nz.at[pcs[st]], sem_s).wait()

    def fcompute(st):
        gx, gy, gz = gxs[st], gys[st], gzs[st]
        hx, hy, hz = hxs[st], hys[st], hzs[st]
        kx, ky, kz = kxs[st], kys[st], kzs[st]
        vx, vy, vz = vxs[st], vys[st], vzs[st]

        def grp(i, carry):
            sl = pl.ds(i * 16, 16)
            ax = gx[sl]
            ay = gy[sl]
            az = gz[sl]
            ux = hx[sl] - ax
            uy = hy[sl] - ay
            uz = hz[sl] - az
            wx = kx[sl] - ax
            wy = ky[sl] - ay
            wz = kz[sl] - az
            cx = uy * wz - uz * wy
            cy = uz * wx - ux * wz
            cz = ux * wy - uy * wx
            r = _rsqrt_sc(cx * cx + cy * cy + cz * cz)
            vx[sl] = cx * r
            vy[sl] = cy * r
            vz[sl] = cz * r
            return carry

        lax.fori_loop(0, CF // 16, grp, 0)

    fload(0, 0)
    fg_fire(0)
    fload(1, 1)
    fg_drain(0)
    fg_fire(1)
    fcompute(0)
    fs_fire(0)
    fload(2, 2)
    fg_drain(1)
    fg_fire(2)
    fcompute(1)
    fs_fire(1)
    fs_drain(0)

    def fbody(b, carry):
        for k in range(3):
            cc = 3 * b + 2 + k
            cur = (2 + k) % 3
            nxt = k % 3
            prv = (1 + k) % 3
            fload(nxt, cc + 1)
            fg_drain(cur)
            fg_fire(nxt)
            fcompute(cur)
            fs_fire(cur)
            fs_drain(prv)
        return carry

    lax.fori_loop(0, 2, fbody, 0)
    _face_last_load(ft_h, w, fvs[0], (pas[0], pbs[0], pcs[0]))
    fg_drain(2)
    fg_fire(0)
    fcompute(2)
    fs_fire(2)
    fs_drain(1)
    fg_drain(0)
    fcompute(0)
    fs_fire(0)
    fs_drain(2)
    fs_drain(0)

    plsc.subcore_barrier()
    o = c * NPAD + off
    for ssrc, hdst in ((sp_t, t_o), (sp_nx, nxo), (sp_ny, nyo), (sp_nz, nzo)):
        pltpu.sync_copy(ssrc.at[pl.ds(off, SL)], zerov)
        pltpu.sync_copy(zerov, hdst.at[pl.ds(o, SL)])


_a_scratch = (
    [pltpu.VMEM((2, CE), _I32) for _ in range(3)]
    + [pltpu.VMEM((CE,), _I32) for _ in range(6)]
    + [pltpu.VMEM((CE,), _F32) for _ in range(3)]
    + [pltpu.VMEM((3, CF), _I32) for _ in range(3)]
    + [pltpu.VMEM((CF,), _I32) for _ in range(9)]
    + [pltpu.VMEM((CF,), _F32) for _ in range(36)]
    + [pltpu.VMEM((SL,), _F32)]
    + [pltpu.VMEM_SHARED((NPAD,), _F32) for _ in range(8)]
    + [pltpu.SemaphoreType.DMA, pltpu.SemaphoreType.DMA]
)

_a_call = functools.partial(
    pl.kernel,
    out_type=tuple(jax.ShapeDtypeStruct((NC * NPAD,), _F32) for _ in range(4)),
    mesh=_MESH,
    scratch_types=_a_scratch,
)(_a_body)


def _d0_body(deg_ref, xx, xy, xz, w1, dis_o, z_o):
    deg = deg_ref[0] + deg_ref[1]
    dis = jnp.where(deg > 0, 1.0 / jnp.sqrt(jnp.maximum(deg, 1e-12)), 0.0)
    sv = xx[...] * w1[0] + xy[...] * w1[1] + xz[...] * w1[2]
    dis_o[...] = dis
    z_o[...] = dis * sv


_d0_call = pl.pallas_call(
    _d0_body,
    out_shape=(jax.ShapeDtypeStruct((RB, 128), _F32),
               jax.ShapeDtypeStruct((RB, 128), _F32)),
    in_specs=[
        pl.BlockSpec(memory_space=pltpu.VMEM),
        pl.BlockSpec(memory_space=pltpu.VMEM),
        pl.BlockSpec(memory_space=pltpu.VMEM),
        pl.BlockSpec(memory_space=pltpu.VMEM),
        pl.BlockSpec(memory_space=pltpu.SMEM),
    ],
)


def _u_body(tp, npx, npy, npz, px, py, pz, xx, xy, xz, dx, dy, dz, dis,
            w0, bb, w1n,
            dxo, dyo, dzo, pxo, pyo, pzo, zo):
    nx = npx[0] + npx[1]
    ny = npy[0] + npy[1]
    nz = npz[0] + npz[1]
    nrm = jnp.sqrt(nx * nx + ny * ny + nz * nz)
    inv = 1.0 / jnp.maximum(nrm, 1e-12)
    nx = nx * inv
    ny = ny * inv
    nz = nz * inv
    acc = tp[0] + tp[1]
    dpn = (px[...] * w0[0] + py[...] * w0[1] + pz[...] * w0[2]
           - dis[...] * acc + bb[0])
    ndx = dx[...] + dpn * nx
    ndy = dy[...] + dpn * ny
    ndz = dz[...] + dpn * nz
    npx_ = xx[...] + ndx
    npy_ = xy[...] + ndy
    npz_ = xz[...] + ndz
    dxo[...] = ndx
    dyo[...] = ndy
    dzo[...] = ndz
    pxo[...] = npx_
    pyo[...] = npy_
    pzo[...] = npz_
    zo[...] = dis[...] * (npx_ * w1n[0] + npy_ * w1n[1] + npz_ * w1n[2])


_u_call = pl.pallas_call(
    _u_body,
    out_shape=tuple(jax.ShapeDtypeStruct((RB, 128), _F32) for _ in range(7)),
    in_specs=(
        [pl.BlockSpec(memory_space=pltpu.VMEM) for _ in range(14)]
        + [pl.BlockSpec(memory_space=pltpu.SMEM) for _ in range(3)]
    ),
)


@jax.jit
def kernel(x, edge_index, faces, W1, b1, W2, b2, W3, b3):
    ei = edge_index.astype(_I32)
    ft = jnp.pad(faces.astype(_I32), ((0, FTPAD), (0, 0))).T

    xx = jnp.pad(x[:, 0], (0, NPAD - N)).reshape(RB, 128)
    xy = jnp.pad(x[:, 1], (0, NPAD - N)).reshape(RB, 128)
    xz = jnp.pad(x[:, 2], (0, NPAD - N)).reshape(RB, 128)

    degp = _p_call(ei).reshape(NC, RB, 128)
    dis, z = _d0_call(degp, xx, xy, xz, W1[1, :, 0])

    px, py, pz = xx, xy, xz
    dx = jnp.zeros((RB, 128), _F32)
    dy = jnp.zeros((RB, 128), _F32)
    dz = jnp.zeros((RB, 128), _F32)
    params = [(W1, b1), (W2, b2), (W3, b3)]
    for i in range(3):
        tp, npx, npy, npz = _a_call(
            ei, ft,
            px.reshape(NPAD), py.reshape(NPAD), pz.reshape(NPAD),
            z.reshape(NPAD))
        w0 = params[i][0][0, :, 0]
        bb = params[i][1]
        w1n = params[(i + 1) % 3][0][1, :, 0]
        dx, dy, dz, px, py, pz, z = _u_call(
            tp.reshape(NC, RB, 128), npx.reshape(NC, RB, 128),
            npy.reshape(NC, RB, 128), npz.reshape(NC, RB, 128),
            px, py, pz, xx, xy, xz, dx, dy, dz, dis,
            w0, bb, w1n)

    dpos = jnp.stack(
        [dx.reshape(NPAD)[:N], dy.reshape(NPAD)[:N], dz.reshape(NPAD)[:N]],
        axis=1)
    return (dpos, dpos, dpos)
